# baseline (speedup 1.0000x reference)
import jax
import jax.numpy as jnp
from jax.experimental import pallas as pl
from jax.experimental.pallas import tpu as pltpu

_NUM_HEADS = 16
_DIM = 64
_K = 1024
_BLOCK_M = 256


def _vq_assign_kernel(x_ref, m_ref, out_ref):
    cols = []
    for hh in range(_NUM_HEADS):
        x = x_ref[:, hh, :]
        m = m_ref[hh]
        cross = jax.lax.dot_general(
            x, m, (((1,), (1,)), ((), ())),
            preferred_element_type=jnp.float32,
        )
        x_sq = jnp.sum(x * x, axis=1, keepdims=True)
        m_sq = jnp.sum(m * m, axis=1)
        dists = x_sq - 2.0 * cross + m_sq[None, :]
        minval = jnp.min(dists, axis=1, keepdims=True)
        iota = jax.lax.broadcasted_iota(jnp.int32, dists.shape, 1)
        idx = jnp.min(jnp.where(dists == minval, iota, _K), axis=1)
        cols.append(idx.astype(jnp.int32))
    out_ref[...] = jnp.stack(cols, axis=1)


@jax.jit
def kernel(x, means):
    b, n, feat = x.shape
    h = _NUM_HEADS
    bn = b * n
    x3 = x.reshape(bn, h, _DIM)
    grid = (bn // _BLOCK_M,)
    out = pl.pallas_call(
        _vq_assign_kernel,
        grid=grid,
        in_specs=[
            pl.BlockSpec((_BLOCK_M, h, _DIM), lambda i: (i, 0, 0)),
            pl.BlockSpec((h, _K, _DIM), lambda i: (0, 0, 0)),
        ],
        out_specs=pl.BlockSpec((_BLOCK_M, h), lambda i: (i, 0)),
        out_shape=jax.ShapeDtypeStruct((bn, h), jnp.int32),
        compiler_params=pltpu.CompilerParams(
            dimension_semantics=("arbitrary",),
        ),
    )(x3, means)
    return out.reshape(b, n, h)


# speedup vs baseline: 1.6228x; 1.6228x over previous
import jax
import jax.numpy as jnp
from jax.experimental import pallas as pl
from jax.experimental.pallas import tpu as pltpu

_NUM_HEADS = 16
_DIM = 64
_K = 1024
_BLOCK_M = 256


def _vq_assign_kernel(x_ref, ma_ref, msq_ref, out_ref):
    cols = []
    for hh in range(_NUM_HEADS):
        x = x_ref[:, hh, :]
        ma = ma_ref[hh]
        cross2 = jax.lax.dot_general(
            x, ma, (((1,), (1,)), ((), ())),
            preferred_element_type=jnp.float32,
        )
        dists = cross2 + msq_ref[hh][None, :]
        minval = jnp.min(dists, axis=1, keepdims=True)
        iota = jax.lax.broadcasted_iota(jnp.int32, dists.shape, 1)
        idx = jnp.min(jnp.where(dists == minval, iota, _K), axis=1)
        cols.append(idx.astype(jnp.int32))
    out_ref[...] = jnp.stack(cols, axis=1)


@jax.jit
def kernel(x, means):
    b, n, feat = x.shape
    h = _NUM_HEADS
    bn = b * n
    x3 = x.reshape(bn, h, _DIM)
    ma = means * -2.0
    msq = jnp.sum(means * means, axis=-1)
    grid = (bn // _BLOCK_M,)
    out = pl.pallas_call(
        _vq_assign_kernel,
        grid=grid,
        in_specs=[
            pl.BlockSpec((_BLOCK_M, h, _DIM), lambda i: (i, 0, 0)),
            pl.BlockSpec((h, _K, _DIM), lambda i: (0, 0, 0)),
            pl.BlockSpec((h, _K), lambda i: (0, 0)),
        ],
        out_specs=pl.BlockSpec((_BLOCK_M, h), lambda i: (i, 0)),
        out_shape=jax.ShapeDtypeStruct((bn, h), jnp.int32),
        compiler_params=pltpu.CompilerParams(
            dimension_semantics=("arbitrary",),
        ),
    )(x3, ma, msq)
    return out.reshape(b, n, h)
